# baseline (speedup 1.0000x reference)
import jax
import jax.numpy as jnp
from jax import lax
from jax.experimental import pallas as pl
from jax.experimental.pallas import tpu as pltpu
from jax.experimental.pallas import tpu_sc as plsc

N_LANES = 16
NUM_CORES = 2
NUM_SUBCORES = 16
NW = NUM_CORES * NUM_SUBCORES
BATCH_SIZE = 16384
DIM = 32
ROWS_PER_W = BATCH_SIZE // NW
CHUNK = 128
NCHUNK = ROWS_PER_W // CHUNK
GROUPS = ROWS_PER_W // N_LANES


def _gmf_body(uidx_hbm, iidx_hbm, ut_hbm, it_hbm, wb_hbm, b_hbm, out_hbm,
              idxu_v, idxi_v, ue, ie, out_v, wb_v, b_v, sem):
    c = lax.axis_index("c")
    s = lax.axis_index("s")
    wid = s * NUM_CORES + c
    base_row = wid * ROWS_PER_W

    pltpu.sync_copy(uidx_hbm.at[pl.ds(base_row, ROWS_PER_W)], idxu_v)
    pltpu.sync_copy(iidx_hbm.at[pl.ds(base_row, ROWS_PER_W)], idxi_v)
    pltpu.sync_copy(wb_hbm, wb_v)
    pltpu.sync_copy(b_hbm, b_v)

    def fire(g, carry):
        base = pl.multiple_of(g * N_LANES, N_LANES)
        uvec = idxu_v[pl.ds(base, N_LANES)]
        ivec = idxi_v[pl.ds(base, N_LANES)]
        row0 = g * DIM

        def fire_d(d, inner):
            pltpu.async_copy(ut_hbm.at[d].at[uvec], ue.at[row0 + d], sem)
            pltpu.async_copy(it_hbm.at[d].at[ivec], ie.at[row0 + d], sem)
            return inner

        lax.fori_loop(0, DIM, fire_d, 0)
        return carry

    lax.fori_loop(0, GROUPS, fire, 0)

    pltpu.make_async_copy(
        ut_hbm.at[pl.ds(0, GROUPS * DIM), pl.ds(0, N_LANES)], ue, sem).wait()
    pltpu.make_async_copy(
        it_hbm.at[pl.ds(0, GROUPS * DIM), pl.ds(0, N_LANES)], ie, sem).wait()

    wvecs = [wb_v[pl.ds(d * N_LANES, N_LANES)] for d in range(DIM)]
    bias = b_v[...]

    def group(g, carry):
        base = pl.multiple_of(g * N_LANES, N_LANES)
        row0 = g * DIM
        acc = bias
        for d in range(DIM):
            uv = ue[row0 + d, :]
            iv = ie[row0 + d, :]
            acc = acc + uv * iv * wvecs[d]
        rating = 1.0 / (1.0 + jnp.exp(-acc))
        out_v[pl.ds(base, N_LANES)] = rating
        return carry

    lax.fori_loop(0, GROUPS, group, 0)
    pltpu.sync_copy(out_v, out_hbm.at[pl.ds(base_row, ROWS_PER_W)])


def kernel(user_indices, item_indices, user_table, item_table, W, b):
    uidx = user_indices.astype(jnp.int32)
    iidx = item_indices.astype(jnp.int32)
    ut_t = user_table.T
    it_t = item_table.T
    wb = jnp.broadcast_to(W.reshape(DIM, 1), (DIM, N_LANES)).reshape(DIM * N_LANES)
    b16 = jnp.broadcast_to(b.reshape(1), (N_LANES,))

    mesh = plsc.VectorSubcoreMesh(core_axis_name="c", subcore_axis_name="s")
    out = pl.kernel(
        _gmf_body,
        out_type=jax.ShapeDtypeStruct((BATCH_SIZE,), jnp.float32),
        mesh=mesh,
        compiler_params=pltpu.CompilerParams(use_tc_tiling_on_sc=False),
        scratch_types=[
            pltpu.VMEM((ROWS_PER_W,), jnp.int32),
            pltpu.VMEM((ROWS_PER_W,), jnp.int32),
            pltpu.VMEM((GROUPS * DIM, N_LANES), jnp.float32),
            pltpu.VMEM((GROUPS * DIM, N_LANES), jnp.float32),
            pltpu.VMEM((ROWS_PER_W,), jnp.float32),
            pltpu.VMEM((DIM * N_LANES,), jnp.float32),
            pltpu.VMEM((N_LANES,), jnp.float32),
            pltpu.SemaphoreType.DMA,
        ],
    )(uidx, iidx, ut_t, it_t, wb, b16)
    return out.reshape(BATCH_SIZE, 1)


# speedup vs baseline: 1.0021x; 1.0021x over previous
import jax
import jax.numpy as jnp
from jax import lax
from jax.experimental import pallas as pl
from jax.experimental.pallas import tpu as pltpu
from jax.experimental.pallas import tpu_sc as plsc

N_LANES = 16
NUM_CORES = 2
NUM_SUBCORES = 16
NW = NUM_CORES * NUM_SUBCORES
BATCH_SIZE = 16384
DIM = 32
ROWS_PER_W = BATCH_SIZE // NW
CHUNK = 128
NCHUNK = ROWS_PER_W // CHUNK
GROUPS = ROWS_PER_W // N_LANES


def _gmf_body(uidx_hbm, iidx_hbm, ut_hbm, it_hbm, wb_hbm, b_hbm, out_hbm,
              idxu_v, idxi_v, ue, ie, out_v, wb_v, b_v, sem):
    c = lax.axis_index("c")
    s = lax.axis_index("s")
    wid = s * NUM_CORES + c
    base_row = wid * ROWS_PER_W

    pltpu.sync_copy(uidx_hbm.at[pl.ds(base_row, ROWS_PER_W)], idxu_v)
    pltpu.sync_copy(iidx_hbm.at[pl.ds(base_row, ROWS_PER_W)], idxi_v)
    pltpu.sync_copy(wb_hbm, wb_v)
    pltpu.sync_copy(b_hbm, b_v)

    def fire(g, carry):
        base = pl.multiple_of(g * N_LANES, N_LANES)
        uvec = idxu_v[pl.ds(base, N_LANES)]
        ivec = idxi_v[pl.ds(base, N_LANES)]
        row0 = g * DIM

        def fire_d(d, inner):
            pltpu.async_copy(
                ut_hbm.at[d].at[plsc.Indices(uvec, ignored_value=-1)],
                ue.at[row0 + d], sem)
            pltpu.async_copy(
                it_hbm.at[d].at[plsc.Indices(ivec, ignored_value=-1)],
                ie.at[row0 + d], sem)
            return inner

        lax.fori_loop(0, DIM, fire_d, 0)
        return carry

    lax.fori_loop(0, GROUPS, fire, 0)

    pltpu.make_async_copy(
        ut_hbm.at[pl.ds(0, GROUPS * DIM), pl.ds(0, N_LANES)], ue, sem).wait()
    pltpu.make_async_copy(
        it_hbm.at[pl.ds(0, GROUPS * DIM), pl.ds(0, N_LANES)], ie, sem).wait()

    wvecs = [wb_v[pl.ds(d * N_LANES, N_LANES)] for d in range(DIM)]
    bias = b_v[...]

    def group(g, carry):
        base = pl.multiple_of(g * N_LANES, N_LANES)
        row0 = g * DIM
        acc = bias
        for d in range(DIM):
            uv = ue[row0 + d, :]
            iv = ie[row0 + d, :]
            acc = acc + uv * iv * wvecs[d]
        rating = 1.0 / (1.0 + jnp.exp(-acc))
        out_v[pl.ds(base, N_LANES)] = rating
        return carry

    lax.fori_loop(0, GROUPS, group, 0)
    pltpu.sync_copy(out_v, out_hbm.at[pl.ds(base_row, ROWS_PER_W)])


def kernel(user_indices, item_indices, user_table, item_table, W, b):
    uidx = user_indices.astype(jnp.int32)
    iidx = item_indices.astype(jnp.int32)
    ut_t = user_table.T
    it_t = item_table.T
    wb = jnp.broadcast_to(W.reshape(DIM, 1), (DIM, N_LANES)).reshape(DIM * N_LANES)
    b16 = jnp.broadcast_to(b.reshape(1), (N_LANES,))

    mesh = plsc.VectorSubcoreMesh(core_axis_name="c", subcore_axis_name="s")
    out = pl.kernel(
        _gmf_body,
        out_type=jax.ShapeDtypeStruct((BATCH_SIZE,), jnp.float32),
        mesh=mesh,
        compiler_params=pltpu.CompilerParams(use_tc_tiling_on_sc=False),
        scratch_types=[
            pltpu.VMEM((ROWS_PER_W,), jnp.int32),
            pltpu.VMEM((ROWS_PER_W,), jnp.int32),
            pltpu.VMEM((GROUPS * DIM, N_LANES), jnp.float32),
            pltpu.VMEM((GROUPS * DIM, N_LANES), jnp.float32),
            pltpu.VMEM((ROWS_PER_W,), jnp.float32),
            pltpu.VMEM((DIM * N_LANES,), jnp.float32),
            pltpu.VMEM((N_LANES,), jnp.float32),
            pltpu.SemaphoreType.DMA,
        ],
    )(uidx, iidx, ut_t, it_t, wb, b16)
    return out.reshape(BATCH_SIZE, 1)


# speedup vs baseline: 5.6549x; 5.6430x over previous
import jax
import jax.numpy as jnp
from jax import lax
from jax.experimental import pallas as pl
from jax.experimental.pallas import tpu as pltpu
from jax.experimental.pallas import tpu_sc as plsc

N_LANES = 16
NUM_CORES = 2
NUM_SUBCORES = 16
NW = NUM_CORES * NUM_SUBCORES
BATCH_SIZE = 16384
DIM = 32
ROWS_PER_W = BATCH_SIZE // NW
CHUNK = 128
NCHUNK = ROWS_PER_W // CHUNK
GROUPS = ROWS_PER_W // N_LANES


def _gmf_body(uidx_hbm, iidx_hbm, ut_hbm, it_hbm, wb_hbm, b_hbm, out_hbm,
              idxu_v, idxi_v, u_rows, i_rows, out_v, wb_v, b_v, sem):
    c = lax.axis_index("c")
    s = lax.axis_index("s")
    wid = s * NUM_CORES + c

    pltpu.sync_copy(uidx_hbm.at[pl.ds(wid * NCHUNK, NCHUNK)], idxu_v)
    pltpu.sync_copy(iidx_hbm.at[pl.ds(wid * NCHUNK, NCHUNK)], idxi_v)
    pltpu.sync_copy(wb_hbm, wb_v)
    pltpu.sync_copy(b_hbm, b_v)

    copies = []
    for j in range(NCHUNK):
        copies.append(pltpu.async_copy(
            ut_hbm.at[idxu_v.at[j]], u_rows.at[pl.ds(j * CHUNK, CHUNK)], sem))
        copies.append(pltpu.async_copy(
            it_hbm.at[idxi_v.at[j]], i_rows.at[pl.ds(j * CHUNK, CHUNK)], sem))
    for cp in copies:
        cp.wait()

    lane_iota = lax.iota(jnp.int32, N_LANES)
    wvecs = [wb_v[d, :] for d in range(DIM)]
    cols = [jnp.full((N_LANES,), d, jnp.int32) for d in range(DIM)]
    bias = b_v[...]

    def group(g, carry):
        base = pl.multiple_of(g * N_LANES, N_LANES)
        rows = base + lane_iota
        acc = bias
        for d in range(DIM):
            uv = plsc.load_gather(u_rows, [rows, cols[d]])
            iv = plsc.load_gather(i_rows, [rows, cols[d]])
            acc = acc + uv * iv * wvecs[d]
        rating = 1.0 / (1.0 + jnp.exp(-acc))
        out_v[pl.ds(base, N_LANES)] = rating
        return carry

    lax.fori_loop(0, GROUPS, group, 0)
    pltpu.sync_copy(out_v, out_hbm.at[pl.ds(wid * ROWS_PER_W, ROWS_PER_W)])


def kernel(user_indices, item_indices, user_table, item_table, W, b):
    uidx = user_indices.astype(jnp.int32).reshape(NW * NCHUNK, CHUNK)
    iidx = item_indices.astype(jnp.int32).reshape(NW * NCHUNK, CHUNK)
    wb = jnp.broadcast_to(W.reshape(DIM, 1), (DIM, N_LANES))
    b16 = jnp.broadcast_to(b.reshape(1), (N_LANES,))

    mesh = plsc.VectorSubcoreMesh(core_axis_name="c", subcore_axis_name="s")
    out = pl.kernel(
        _gmf_body,
        out_type=jax.ShapeDtypeStruct((BATCH_SIZE,), jnp.float32),
        mesh=mesh,
        compiler_params=pltpu.CompilerParams(
            needs_layout_passes=False, use_tc_tiling_on_sc=False),
        scratch_types=[
            pltpu.VMEM((NCHUNK, CHUNK), jnp.int32),
            pltpu.VMEM((NCHUNK, CHUNK), jnp.int32),
            pltpu.VMEM((ROWS_PER_W, DIM), jnp.float32),
            pltpu.VMEM((ROWS_PER_W, DIM), jnp.float32),
            pltpu.VMEM((ROWS_PER_W,), jnp.float32),
            pltpu.VMEM((DIM, N_LANES), jnp.float32),
            pltpu.VMEM((N_LANES,), jnp.float32),
            pltpu.SemaphoreType.DMA,
        ],
    )(uidx, iidx, user_table, item_table, wb, b16)
    return out.reshape(BATCH_SIZE, 1)
